# baseline (speedup 1.0000x reference)
import jax
import jax.numpy as jnp
from jax.experimental import pallas as pl

NUM_EMB = 512
EMB_DIM = 64
TOPK = 4
PIX_BLOCK = 2048


def _vq_kernel(xp_ref, embT_ref, out_st_ref, out_q_ref):
    xp = xp_ref[...]
    embT = embT_ref[...]
    P = xp.shape[0]
    xT = xp.T

    hi = embT.astype(jnp.bfloat16)
    r1 = embT - hi.astype(jnp.float32)
    mid = r1.astype(jnp.bfloat16)
    lo = (r1 - mid.astype(jnp.float32)).astype(jnp.bfloat16)
    esplit = jnp.concatenate([hi, mid, lo], axis=0)

    en = jnp.sum(embT * embT, axis=0, keepdims=True)
    eaug = jnp.concatenate([embT, en], axis=0)
    xaug = jnp.concatenate(
        [xT * jnp.float32(-2.0), jnp.ones((1, P), jnp.float32)], axis=0)
    d2m = jax.lax.dot_general(
        eaug, xaug, (((0,), (0,)), ((), ())),
        preferred_element_type=jnp.float32)

    kb = jax.lax.bitcast_convert_type(d2m, jnp.int32)
    key = kb ^ jax.lax.shift_right_logical(
        jax.lax.shift_right_arithmetic(kb, 31), 1)
    sub = jax.lax.broadcasted_iota(jnp.int32, (NUM_EMB, P), 0)
    work = (key & jnp.int32(~511)) | sub

    ohs, cand_is = [], []
    for k in range(TOPK):
        m = jnp.min(work, axis=0, keepdims=True)
        eq = work == m
        ohs.append(jnp.where(eq, jnp.float32(1.0),
                             jnp.float32(0.0)).astype(jnp.bfloat16))
        cand_is.append(m & jnp.int32(511))
        if k + 1 < TOPK:
            work = jnp.where(eq, jnp.int32(0x7FFFFFFF), work)

    oh_all = jnp.concatenate(ohs, axis=1)
    g3 = jnp.dot(esplit, oh_all,
                 preferred_element_type=jnp.float32)
    g_all = (g3[0:EMB_DIM] + g3[EMB_DIM:2 * EMB_DIM]) \
        + g3[2 * EMB_DIM:3 * EMB_DIM]

    x_all = jnp.concatenate([xT] * TOPK, axis=1)
    acc = jnp.zeros((1, TOPK * P), jnp.float32)
    for c in range(EMB_DIM):
        t = x_all[c:c + 1, :] - g_all[c:c + 1, :]
        acc = acc + t * t
    d_all = jnp.sqrt(acc)

    best_d = None
    for k in range(TOPK):
        d = d_all[:, k * P:(k + 1) * P]
        gT = g_all[:, k * P:(k + 1) * P]
        cand_i = cand_is[k]
        if best_d is None:
            best_d, best_i, best_g = d, cand_i, gT
        else:
            better = (d < best_d) | ((d == best_d) & (cand_i < best_i))
            best_d = jnp.where(better, d, best_d)
            best_i = jnp.where(better, cand_i, best_i)
            best_g = jnp.where(jnp.broadcast_to(better, gT.shape), gT, best_g)

    g_pc = best_g.T
    out_q_ref[...] = g_pc
    out_st_ref[...] = xp + (g_pc - xp)


def kernel(inputs, embedding):
    B, C, H, W = inputs.shape
    P = B * H * W
    xp = inputs.transpose(0, 2, 3, 1).reshape(P, C)
    embT = embedding.T
    out_st, out_q = pl.pallas_call(
        _vq_kernel,
        grid=(P // PIX_BLOCK,),
        in_specs=[
            pl.BlockSpec((PIX_BLOCK, C), lambda b: (b, 0)),
            pl.BlockSpec((C, NUM_EMB), lambda b: (0, 0)),
        ],
        out_specs=[
            pl.BlockSpec((PIX_BLOCK, C), lambda b: (b, 0)),
            pl.BlockSpec((PIX_BLOCK, C), lambda b: (b, 0)),
        ],
        out_shape=[
            jax.ShapeDtypeStruct((P, C), jnp.float32),
            jax.ShapeDtypeStruct((P, C), jnp.float32),
        ],
    )(xp, embT)
    out_st = out_st.reshape(B, H, W, C).transpose(0, 3, 1, 2)
    out_q = out_q.reshape(B, H, W, C).transpose(0, 3, 1, 2)
    return (out_st, out_q)


# speedup vs baseline: 1.1005x; 1.1005x over previous
import jax
import jax.numpy as jnp
from jax.experimental import pallas as pl

NUM_EMB = 512
EMB_DIM = 64
TOPK = 4
PIX_BLOCK = 1024


def _vq_kernel(xp_ref, embT_ref, out_st_ref, out_q_ref):
    xp = xp_ref[...]
    embT = embT_ref[...]
    P = xp.shape[0]
    xT = xp.T

    hi = embT.astype(jnp.bfloat16)
    r1 = embT - hi.astype(jnp.float32)
    mid = r1.astype(jnp.bfloat16)
    lo = (r1 - mid.astype(jnp.float32)).astype(jnp.bfloat16)
    esplit = jnp.concatenate([hi, mid, lo], axis=0)

    en = jnp.sum(embT * embT, axis=0, keepdims=True)
    eaug = jnp.concatenate([embT, en], axis=0)
    xaug = jnp.concatenate(
        [xT * jnp.float32(-2.0), jnp.ones((1, P), jnp.float32)], axis=0)
    d2m = jax.lax.dot_general(
        eaug, xaug, (((0,), (0,)), ((), ())),
        preferred_element_type=jnp.float32)

    kb = jax.lax.bitcast_convert_type(d2m, jnp.int32)
    key = kb ^ jax.lax.shift_right_logical(
        jax.lax.shift_right_arithmetic(kb, 31), 1)
    sub = jax.lax.broadcasted_iota(jnp.int32, (NUM_EMB, P), 0)
    work = (key & jnp.int32(~511)) | sub

    q = EMB_DIM * 2
    s0, s1, s2, s3 = (work[0 * q:1 * q], work[1 * q:2 * q],
                      work[2 * q:3 * q], work[3 * q:4 * q])
    s0, s1 = jnp.minimum(s0, s1), jnp.maximum(s0, s1)
    s2, s3 = jnp.minimum(s2, s3), jnp.maximum(s2, s3)
    s0, s2 = jnp.minimum(s0, s2), jnp.maximum(s0, s2)
    s1, s3 = jnp.minimum(s1, s3), jnp.maximum(s1, s3)
    s1, s2 = jnp.minimum(s1, s2), jnp.maximum(s1, s2)

    ohs, cand_is = [], []
    for k in range(TOPK):
        m = jnp.min(s0, axis=0, keepdims=True)
        eq = work == m
        ohs.append(jnp.where(eq, jnp.float32(1.0),
                             jnp.float32(0.0)).astype(jnp.bfloat16))
        cand_is.append(m & jnp.int32(511))
        if k + 1 < TOPK:
            qeq = s0 == m
            s0 = jnp.where(qeq, s1, s0)
            s1 = jnp.where(qeq, s2, s1)
            s2 = jnp.where(qeq, s3, s2)
            s3 = jnp.where(qeq, jnp.int32(0x7FFFFFFF), s3)

    oh_all = jnp.concatenate(ohs, axis=1)
    g3 = jnp.dot(esplit, oh_all,
                 preferred_element_type=jnp.float32)
    g_all = (g3[0:EMB_DIM] + g3[EMB_DIM:2 * EMB_DIM]) \
        + g3[2 * EMB_DIM:3 * EMB_DIM]

    x_all = jnp.concatenate([xT] * TOPK, axis=1)
    acc = jnp.zeros((1, TOPK * P), jnp.float32)
    for c in range(EMB_DIM):
        t = x_all[c:c + 1, :] - g_all[c:c + 1, :]
        acc = acc + t * t
    d_all = jnp.sqrt(acc)

    best_d = None
    for k in range(TOPK):
        d = d_all[:, k * P:(k + 1) * P]
        gT = g_all[:, k * P:(k + 1) * P]
        cand_i = cand_is[k]
        if best_d is None:
            best_d, best_i, best_g = d, cand_i, gT
        else:
            better = (d < best_d) | ((d == best_d) & (cand_i < best_i))
            best_d = jnp.where(better, d, best_d)
            best_i = jnp.where(better, cand_i, best_i)
            best_g = jnp.where(jnp.broadcast_to(better, gT.shape), gT, best_g)

    g_pc = best_g.T
    out_q_ref[...] = g_pc
    out_st_ref[...] = xp + (g_pc - xp)


def kernel(inputs, embedding):
    B, C, H, W = inputs.shape
    P = B * H * W
    xp = inputs.transpose(0, 2, 3, 1).reshape(P, C)
    embT = embedding.T
    out_st, out_q = pl.pallas_call(
        _vq_kernel,
        grid=(P // PIX_BLOCK,),
        in_specs=[
            pl.BlockSpec((PIX_BLOCK, C), lambda b: (b, 0)),
            pl.BlockSpec((C, NUM_EMB), lambda b: (0, 0)),
        ],
        out_specs=[
            pl.BlockSpec((PIX_BLOCK, C), lambda b: (b, 0)),
            pl.BlockSpec((PIX_BLOCK, C), lambda b: (b, 0)),
        ],
        out_shape=[
            jax.ShapeDtypeStruct((P, C), jnp.float32),
            jax.ShapeDtypeStruct((P, C), jnp.float32),
        ],
    )(xp, embT)
    out_st = out_st.reshape(B, H, W, C).transpose(0, 3, 1, 2)
    out_q = out_q.reshape(B, H, W, C).transpose(0, 3, 1, 2)
    return (out_st, out_q)
